# speedup vs baseline: 13.2438x; 13.2438x over previous
import jax
import jax.numpy as jnp
from jax import lax
from jax.experimental import pallas as pl
from jax.experimental.pallas import tpu as pltpu
from jax.experimental.pallas import tpu_sc as plsc

N = 10000
NP = 10240
D = 128
NB = 64
E = 320000
NC, NS = 2, 16
NW = NC * NS
CK = 128
CH = 79
EP = NW * CH * CK
RPT = NP // NS
DUMMY = N
RB = 512
GRID = NP // RB

_MESH = plsc.VectorSubcoreMesh(core_axis_name="c", subcore_axis_name="s",
                               num_cores=NC, num_subcores=NS)


def _deg_body(dst_hbm, cnt_out, dst_v, ones_v, zb_v, cnt_sp):
    c = lax.axis_index("c")
    s = lax.axis_index("s")
    w = c * NS + s
    pltpu.sync_copy(dst_hbm.at[w], dst_v)
    z16 = jnp.zeros((16,), jnp.float32)
    o16 = jnp.ones((16,), jnp.float32)
    for i in range(RPT // 16):
        zb_v[pl.ds(16 * i, 16)] = z16
    for i in range(CK // 16):
        ones_v[pl.ds(16 * i, 16)] = o16
    pltpu.sync_copy(zb_v, cnt_sp.at[pl.ds(s * RPT, RPT)])
    plsc.subcore_barrier()
    for g in range(CH):
        pltpu.sync_copy(ones_v, cnt_sp.at[dst_v.at[g]], add=True)
    plsc.subcore_barrier()
    pltpu.sync_copy(cnt_sp.at[pl.ds(s * RPT, RPT)],
                    cnt_out.at[c, pl.ds(s * RPT, RPT)])


_deg_call = pl.kernel(
    _deg_body,
    out_type=jax.ShapeDtypeStruct((NC, NP), jnp.float32),
    mesh=_MESH,
    scratch_types=[
        pltpu.VMEM((CH, CK), jnp.int32),
        pltpu.VMEM((CK,), jnp.float32),
        pltpu.VMEM((RPT,), jnp.float32),
        pltpu.VMEM_SHARED((NP,), jnp.float32),
    ],
)


def _scat_body(y_hbm, src_hbm, dst_hbm, out_hbm, src_v, dst_v, rows_v, acc_sp,
               sem):
    c = lax.axis_index("c")
    s = lax.axis_index("s")
    w = c * NS + s
    pltpu.sync_copy(src_hbm.at[w], src_v)
    pltpu.sync_copy(dst_hbm.at[w], dst_v)
    base = s * RPT

    @pl.when(c == 0)
    def _():
        pltpu.sync_copy(y_hbm.at[pl.ds(base, RPT)], acc_sp.at[pl.ds(base, RPT)])

    @pl.when(c != 0)
    def _():
        z16 = jnp.zeros((16,), jnp.float32)

        def _zrow(i, carry):
            for j in range(D // 16):
                rows_v[i, pl.ds(16 * j, 16)] = z16
            return carry

        lax.fori_loop(0, CK, _zrow, 0)
        for k in range(RPT // CK):
            pltpu.sync_copy(rows_v, acc_sp.at[pl.ds(base + k * CK, CK)])

    plsc.subcore_barrier()

    for g in range(CH):
        pltpu.async_copy(y_hbm.at[src_v.at[g]], rows_v, sem).wait()
        pltpu.sync_copy(rows_v, acc_sp.at[dst_v.at[g]], add=True)

    plsc.subcore_barrier()
    pltpu.sync_copy(acc_sp.at[pl.ds(base, RPT)],
                    out_hbm.at[c, pl.ds(base, RPT)])


_scat_call = pl.kernel(
    _scat_body,
    out_type=jax.ShapeDtypeStruct((NC, NP, D), jnp.float32),
    mesh=_MESH,
    scratch_types=[
        pltpu.VMEM((CH, CK), jnp.int32),
        pltpu.VMEM((CH, CK), jnp.int32),
        pltpu.VMEM((CK, D), jnp.float32),
        pltpu.VMEM_SHARED((NP, D), jnp.float32),
        pltpu.SemaphoreType.DMA,
    ],
)


def _tc_a_body(cnt0_ref, cnt1_ref, x_ref, w_ref, dis_ref, y_ref):
    deg = cnt0_ref[...] + cnt1_ref[...] + 1.0
    dis = lax.rsqrt(deg)
    dis_ref[...] = dis
    y_ref[...] = jnp.dot(x_ref[...], w_ref[...],
                         preferred_element_type=jnp.float32) * dis


_tc_a = pl.pallas_call(
    _tc_a_body,
    grid=(GRID,),
    in_specs=[
        pl.BlockSpec((RB, 1), lambda i: (i, 0)),
        pl.BlockSpec((RB, 1), lambda i: (i, 0)),
        pl.BlockSpec((RB, D), lambda i: (i, 0)),
        pl.BlockSpec((D, D), lambda i: (0, 0)),
    ],
    out_specs=[
        pl.BlockSpec((RB, 1), lambda i: (i, 0)),
        pl.BlockSpec((RB, D), lambda i: (i, 0)),
    ],
    out_shape=[
        jax.ShapeDtypeStruct((NP, 1), jnp.float32),
        jax.ShapeDtypeStruct((NP, D), jnp.float32),
    ],
)


def _tc_mid_body(a0_ref, a1_ref, dis_ref, b_ref, w_ref, y2_ref):
    dis = dis_ref[...]
    h = jnp.maximum((a0_ref[...] + a1_ref[...]) * dis + b_ref[...], 0.0)
    y2_ref[...] = jnp.dot(h, w_ref[...],
                          preferred_element_type=jnp.float32) * dis


_tc_mid = pl.pallas_call(
    _tc_mid_body,
    grid=(GRID,),
    in_specs=[
        pl.BlockSpec((RB, D), lambda i: (i, 0)),
        pl.BlockSpec((RB, D), lambda i: (i, 0)),
        pl.BlockSpec((RB, 1), lambda i: (i, 0)),
        pl.BlockSpec((1, D), lambda i: (0, 0)),
        pl.BlockSpec((D, D), lambda i: (0, 0)),
    ],
    out_specs=pl.BlockSpec((RB, D), lambda i: (i, 0)),
    out_shape=jax.ShapeDtypeStruct((NP, D), jnp.float32),
)


def _tc_fin_body(a0_ref, a1_ref, dis_ref, b_ref, batch_ref, wfc_ref, bfc_ref,
                 out_ref, sums, cnts):
    i = pl.program_id(0)

    @pl.when(i == 0)
    def _():
        sums[...] = jnp.zeros_like(sums)
        cnts[...] = jnp.zeros_like(cnts)

    h = jnp.maximum((a0_ref[...] + a1_ref[...]) * dis_ref[...] + b_ref[...],
                    0.0)
    seg = lax.broadcasted_iota(jnp.int32, (RB, NB), 1)
    p = (batch_ref[...] == seg).astype(jnp.float32)
    dn = (((0,), (0,)), ((), ()))
    sums[...] += lax.dot_general(p, h, dn, preferred_element_type=jnp.float32)
    cnts[...] += lax.dot_general(p, jnp.ones((RB, D), jnp.float32), dn,
                                 preferred_element_type=jnp.float32)

    @pl.when(i == GRID - 1)
    def _():
        pooled = sums[...] / jnp.maximum(cnts[...], 1.0)
        out_ref[...] = jnp.dot(pooled, wfc_ref[...],
                               preferred_element_type=jnp.float32) + bfc_ref[...]


_tc_fin = pl.pallas_call(
    _tc_fin_body,
    grid=(GRID,),
    in_specs=[
        pl.BlockSpec((RB, D), lambda i: (i, 0)),
        pl.BlockSpec((RB, D), lambda i: (i, 0)),
        pl.BlockSpec((RB, 1), lambda i: (i, 0)),
        pl.BlockSpec((1, D), lambda i: (0, 0)),
        pl.BlockSpec((RB, 1), lambda i: (i, 0)),
        pl.BlockSpec((D, 1), lambda i: (0, 0)),
        pl.BlockSpec((1, 1), lambda i: (0, 0)),
    ],
    out_specs=pl.BlockSpec((NB, 1), lambda i: (0, 0)),
    out_shape=jax.ShapeDtypeStruct((NB, 1), jnp.float32),
    scratch_shapes=[
        pltpu.VMEM((NB, D), jnp.float32),
        pltpu.VMEM((NB, D), jnp.float32),
    ],
)


def kernel(x, edge_index, batch, W1, b1, W2, b2, Wfc, bfc):
    x = x.astype(jnp.float32)
    src = edge_index[0].astype(jnp.int32)
    dst = edge_index[1].astype(jnp.int32)
    pad = jnp.full((EP - E,), DUMMY, jnp.int32)
    srcp = jnp.concatenate([src, pad]).reshape(NW, CH, CK)
    dstp = jnp.concatenate([dst, pad]).reshape(NW, CH, CK)
    xp = jnp.pad(x, ((0, NP - N), (0, 0)))
    batchp = jnp.pad(batch.astype(jnp.int32), (0, NP - N),
                     constant_values=NB).reshape(NP, 1)

    cnt = _deg_call(dstp)
    cnt0 = cnt[0].reshape(NP, 1)
    cnt1 = cnt[1].reshape(NP, 1)
    dis, y1 = _tc_a(cnt0, cnt1, xp, W1)
    p1 = _scat_call(y1, srcp, dstp)
    y2 = _tc_mid(p1[0], p1[1], dis, b1.reshape(1, D), W2)
    p2 = _scat_call(y2, srcp, dstp)
    out = _tc_fin(p2[0], p2[1], dis, b2.reshape(1, D), batchp, Wfc,
                  bfc.reshape(1, 1))
    return out.reshape(NB)


# speedup vs baseline: 15.7892x; 1.1922x over previous
import jax
import jax.numpy as jnp
from jax import lax
from jax.experimental import pallas as pl
from jax.experimental.pallas import tpu as pltpu
from jax.experimental.pallas import tpu_sc as plsc

N = 10000
NP = 10240
D = 128
NB = 64
E = 320000
NC, NS = 2, 16
NW = NC * NS
CK = 128
CH = 79
PH = 40
EP = NW * CH * CK
RPT = NP // NS
DUMMY = N
RB = 512
GRID = NP // RB

_MESH = plsc.VectorSubcoreMesh(core_axis_name="c", subcore_axis_name="s",
                               num_cores=NC, num_subcores=NS)


def _deg_body(dst_hbm, cnt_out, dst_v, ones_v, zb_v, cnt_sp):
    c = lax.axis_index("c")
    s = lax.axis_index("s")
    w = c * NS + s
    pltpu.sync_copy(dst_hbm.at[w], dst_v)
    z16 = jnp.zeros((16,), jnp.float32)
    o16 = jnp.ones((16,), jnp.float32)
    for i in range(RPT // 16):
        zb_v[pl.ds(16 * i, 16)] = z16
    for i in range(CK // 16):
        ones_v[pl.ds(16 * i, 16)] = o16
    pltpu.sync_copy(zb_v, cnt_sp.at[pl.ds(s * RPT, RPT)])
    plsc.subcore_barrier()
    for g in range(CH):
        pltpu.sync_copy(ones_v, cnt_sp.at[dst_v.at[g]], add=True)
    plsc.subcore_barrier()
    pltpu.sync_copy(cnt_sp.at[pl.ds(s * RPT, RPT)],
                    cnt_out.at[c, pl.ds(s * RPT, RPT)])


_deg_call = pl.kernel(
    _deg_body,
    out_type=jax.ShapeDtypeStruct((NC, NP), jnp.float32),
    mesh=_MESH,
    scratch_types=[
        pltpu.VMEM((CH, CK), jnp.int32),
        pltpu.VMEM((CK,), jnp.float32),
        pltpu.VMEM((RPT,), jnp.float32),
        pltpu.VMEM_SHARED((NP,), jnp.float32),
    ],
)


NBUF = 2


def _scat_body(y_hbm, src_hbm, dst_hbm, out_hbm, src_v, dst_v, rows0, rows1,
               acc_sp, gsem0, gsem1):
    rows = (rows0, rows1)
    gsems = (gsem0, gsem1)
    c = lax.axis_index("c")
    s = lax.axis_index("s")
    w = c * NS + s
    base = s * RPT

    @pl.when(c == 0)
    def _():
        pltpu.sync_copy(y_hbm.at[pl.ds(base, RPT)], acc_sp.at[pl.ds(base, RPT)])

    @pl.when(c != 0)
    def _():
        z16 = jnp.zeros((16,), jnp.float32)

        def _zrow(i, carry):
            for j in range(D // 16):
                rows0[i, pl.ds(16 * j, 16)] = z16
            return carry

        lax.fori_loop(0, CK, _zrow, 0)
        off = 0
        while off < RPT:
            n = min(CK, RPT - off)
            pltpu.sync_copy(rows0.at[pl.ds(0, n)],
                            acc_sp.at[pl.ds(base + off, n)])
            off += n

    plsc.subcore_barrier()

    done = 0
    while done < CH:
        n = min(PH, CH - done)
        pltpu.sync_copy(src_hbm.at[w, pl.ds(done, n)], src_v.at[pl.ds(0, n)])
        pltpu.sync_copy(dst_hbm.at[w, pl.ds(done, n)], dst_v.at[pl.ds(0, n)])
        gd = [None] * n
        for g in range(n + 1):
            if g < n:
                b = g % NBUF
                gd[g] = pltpu.async_copy(y_hbm.at[src_v.at[g]], rows[b],
                                         gsems[b])
            if g >= 1:
                h = g - 1
                gd[h].wait()
                pltpu.sync_copy(rows[h % NBUF], acc_sp.at[dst_v.at[h]],
                                add=True)
        done += n

    plsc.subcore_barrier()
    pltpu.sync_copy(acc_sp.at[pl.ds(base, RPT)],
                    out_hbm.at[c, pl.ds(base, RPT)])


_scat_call = pl.kernel(
    _scat_body,
    out_type=jax.ShapeDtypeStruct((NC, NP, D), jnp.float32),
    mesh=_MESH,
    scratch_types=[
        pltpu.VMEM((PH, CK), jnp.int32),
        pltpu.VMEM((PH, CK), jnp.int32),
        pltpu.VMEM((CK, D), jnp.float32),
        pltpu.VMEM((CK, D), jnp.float32),
        pltpu.VMEM_SHARED((NP, D), jnp.float32),
        pltpu.SemaphoreType.DMA,
        pltpu.SemaphoreType.DMA,
    ],
)


def _tc_a_body(cnt0_ref, cnt1_ref, x_ref, w_ref, dis_ref, y_ref):
    deg = cnt0_ref[...] + cnt1_ref[...] + 1.0
    dis = lax.rsqrt(deg)
    dis_ref[...] = dis
    y_ref[...] = jnp.dot(x_ref[...], w_ref[...],
                         preferred_element_type=jnp.float32) * dis


_tc_a = pl.pallas_call(
    _tc_a_body,
    grid=(GRID,),
    in_specs=[
        pl.BlockSpec((RB, 1), lambda i: (i, 0)),
        pl.BlockSpec((RB, 1), lambda i: (i, 0)),
        pl.BlockSpec((RB, D), lambda i: (i, 0)),
        pl.BlockSpec((D, D), lambda i: (0, 0)),
    ],
    out_specs=[
        pl.BlockSpec((RB, 1), lambda i: (i, 0)),
        pl.BlockSpec((RB, D), lambda i: (i, 0)),
    ],
    out_shape=[
        jax.ShapeDtypeStruct((NP, 1), jnp.float32),
        jax.ShapeDtypeStruct((NP, D), jnp.float32),
    ],
)


def _tc_mid_body(a0_ref, a1_ref, dis_ref, b_ref, w_ref, y2_ref):
    dis = dis_ref[...]
    h = jnp.maximum((a0_ref[...] + a1_ref[...]) * dis + b_ref[...], 0.0)
    y2_ref[...] = jnp.dot(h, w_ref[...],
                          preferred_element_type=jnp.float32) * dis


_tc_mid = pl.pallas_call(
    _tc_mid_body,
    grid=(GRID,),
    in_specs=[
        pl.BlockSpec((RB, D), lambda i: (i, 0)),
        pl.BlockSpec((RB, D), lambda i: (i, 0)),
        pl.BlockSpec((RB, 1), lambda i: (i, 0)),
        pl.BlockSpec((1, D), lambda i: (0, 0)),
        pl.BlockSpec((D, D), lambda i: (0, 0)),
    ],
    out_specs=pl.BlockSpec((RB, D), lambda i: (i, 0)),
    out_shape=jax.ShapeDtypeStruct((NP, D), jnp.float32),
)


def _tc_fin_body(a0_ref, a1_ref, dis_ref, b_ref, batch_ref, wfc_ref, bfc_ref,
                 out_ref, sums, cnts):
    i = pl.program_id(0)

    @pl.when(i == 0)
    def _():
        sums[...] = jnp.zeros_like(sums)
        cnts[...] = jnp.zeros_like(cnts)

    h = jnp.maximum((a0_ref[...] + a1_ref[...]) * dis_ref[...] + b_ref[...],
                    0.0)
    seg = lax.broadcasted_iota(jnp.int32, (RB, NB), 1)
    p = (batch_ref[...] == seg).astype(jnp.float32)
    dn = (((0,), (0,)), ((), ()))
    sums[...] += lax.dot_general(p, h, dn, preferred_element_type=jnp.float32)
    cnts[...] += lax.dot_general(p, jnp.ones((RB, D), jnp.float32), dn,
                                 preferred_element_type=jnp.float32)

    @pl.when(i == GRID - 1)
    def _():
        pooled = sums[...] / jnp.maximum(cnts[...], 1.0)
        out_ref[...] = jnp.dot(pooled, wfc_ref[...],
                               preferred_element_type=jnp.float32) + bfc_ref[...]


_tc_fin = pl.pallas_call(
    _tc_fin_body,
    grid=(GRID,),
    in_specs=[
        pl.BlockSpec((RB, D), lambda i: (i, 0)),
        pl.BlockSpec((RB, D), lambda i: (i, 0)),
        pl.BlockSpec((RB, 1), lambda i: (i, 0)),
        pl.BlockSpec((1, D), lambda i: (0, 0)),
        pl.BlockSpec((RB, 1), lambda i: (i, 0)),
        pl.BlockSpec((D, 1), lambda i: (0, 0)),
        pl.BlockSpec((1, 1), lambda i: (0, 0)),
    ],
    out_specs=pl.BlockSpec((NB, 1), lambda i: (0, 0)),
    out_shape=jax.ShapeDtypeStruct((NB, 1), jnp.float32),
    scratch_shapes=[
        pltpu.VMEM((NB, D), jnp.float32),
        pltpu.VMEM((NB, D), jnp.float32),
    ],
)


def kernel(x, edge_index, batch, W1, b1, W2, b2, Wfc, bfc):
    x = x.astype(jnp.float32)
    src = edge_index[0].astype(jnp.int32)
    dst = edge_index[1].astype(jnp.int32)
    pad = jnp.full((EP - E,), DUMMY, jnp.int32)
    srcp = jnp.concatenate([src, pad]).reshape(NW, CH, CK)
    dstp = jnp.concatenate([dst, pad]).reshape(NW, CH, CK)
    xp = jnp.pad(x, ((0, NP - N), (0, 0)))
    batchp = jnp.pad(batch.astype(jnp.int32), (0, NP - N),
                     constant_values=NB).reshape(NP, 1)

    cnt = _deg_call(dstp)
    cnt0 = cnt[0].reshape(NP, 1)
    cnt1 = cnt[1].reshape(NP, 1)
    dis, y1 = _tc_a(cnt0, cnt1, xp, W1)
    p1 = _scat_call(y1, srcp, dstp)
    y2 = _tc_mid(p1[0], p1[1], dis, b1.reshape(1, D), W2)
    p2 = _scat_call(y2, srcp, dstp)
    out = _tc_fin(p2[0], p2[1], dis, b2.reshape(1, D), batchp, Wfc,
                  bfc.reshape(1, 1))
    return out.reshape(NB)


# speedup vs baseline: 29.2155x; 1.8504x over previous
import jax
import jax.numpy as jnp
from jax import lax
from jax.experimental import pallas as pl
from jax.experimental.pallas import tpu as pltpu
from jax.experimental.pallas import tpu_sc as plsc

N = 10000
NP = 10240
D = 128
NB = 64
E = 320000
NC, NS = 2, 16
NW = NC * NS
CK = 128
CH = 79
PH = 40
EP = NW * CH * CK
RPT = NP // NS
DUMMY = N
RB = 512
GRID = NP // RB

_MESH = plsc.VectorSubcoreMesh(core_axis_name="c", subcore_axis_name="s",
                               num_cores=NC, num_subcores=NS)


def _deg_body(dst_hbm, cnt_out, dst_v, ones_v, zb_v, cnt_sp):
    c = lax.axis_index("c")
    s = lax.axis_index("s")
    w = c * NS + s
    pltpu.sync_copy(dst_hbm.at[w], dst_v)
    z16 = jnp.zeros((16,), jnp.float32)
    o16 = jnp.ones((16,), jnp.float32)
    for i in range(RPT // 16):
        zb_v[pl.ds(16 * i, 16)] = z16
    for i in range(CK // 16):
        ones_v[pl.ds(16 * i, 16)] = o16
    pltpu.sync_copy(zb_v, cnt_sp.at[pl.ds(s * RPT, RPT)])
    plsc.subcore_barrier()
    for g in range(CH):
        pltpu.sync_copy(ones_v, cnt_sp.at[dst_v.at[g]], add=True)
    plsc.subcore_barrier()
    pltpu.sync_copy(cnt_sp.at[pl.ds(s * RPT, RPT)],
                    cnt_out.at[c, pl.ds(s * RPT, RPT)])


_deg_call = pl.kernel(
    _deg_body,
    out_type=jax.ShapeDtypeStruct((NC, NP), jnp.float32),
    mesh=_MESH,
    scratch_types=[
        pltpu.VMEM((CH, CK), jnp.int32),
        pltpu.VMEM((CK,), jnp.float32),
        pltpu.VMEM((RPT,), jnp.float32),
        pltpu.VMEM_SHARED((NP,), jnp.float32),
    ],
)


NBUF = 2


def _scat_body(y_hbm, src_hbm, dst_hbm, out_hbm, src_v, dst_v, rows0, rows1,
               acc_sp, gsem0, gsem1):
    rows = (rows0, rows1)
    gsems = (gsem0, gsem1)
    c = lax.axis_index("c")
    s = lax.axis_index("s")
    w = c * NS + s
    base = s * RPT

    @pl.when(c == 0)
    def _():
        pltpu.sync_copy(y_hbm.at[pl.ds(base, RPT)], acc_sp.at[pl.ds(base, RPT)])

    @pl.when(c != 0)
    def _():
        z16 = jnp.zeros((16,), jnp.float32)

        def _zrow(i, carry):
            for j in range(D // 16):
                rows0[i, pl.ds(16 * j, 16)] = z16
            return carry

        lax.fori_loop(0, CK, _zrow, 0)
        off = 0
        while off < RPT:
            n = min(CK, RPT - off)
            pltpu.sync_copy(rows0.at[pl.ds(0, n)],
                            acc_sp.at[pl.ds(base + off, n)])
            off += n

    plsc.subcore_barrier()

    done = 0
    while done < CH:
        n = min(PH, CH - done)
        pltpu.sync_copy(src_hbm.at[w, pl.ds(done, n)], src_v.at[pl.ds(0, n)])
        pltpu.sync_copy(dst_hbm.at[w, pl.ds(done, n)], dst_v.at[pl.ds(0, n)])
        gd = [None] * n
        for g in range(n + 1):
            if g < n:
                b = g % NBUF
                gd[g] = pltpu.async_copy(y_hbm.at[src_v.at[g]], rows[b],
                                         gsems[b])
            if g >= 1:
                h = g - 1
                gd[h].wait()
                pltpu.sync_copy(rows[h % NBUF], acc_sp.at[dst_v.at[h]],
                                add=True)
        done += n

    plsc.subcore_barrier()
    pltpu.sync_copy(acc_sp.at[pl.ds(base, RPT)],
                    out_hbm.at[c, pl.ds(base, RPT)])


_scat_call = pl.kernel(
    _scat_body,
    out_type=jax.ShapeDtypeStruct((NC, NP, D), jnp.float32),
    mesh=_MESH,
    scratch_types=[
        pltpu.VMEM((PH, CK), jnp.int32),
        pltpu.VMEM((PH, CK), jnp.int32),
        pltpu.VMEM((CK, D), jnp.float32),
        pltpu.VMEM((CK, D), jnp.float32),
        pltpu.VMEM_SHARED((NP, D), jnp.float32),
        pltpu.SemaphoreType.DMA,
        pltpu.SemaphoreType.DMA,
    ],
)


def _tc_a_body(cnt0_ref, cnt1_ref, x_ref, w_ref, dis_ref, y_ref):
    deg = cnt0_ref[...] + cnt1_ref[...] + 1.0
    dis = lax.rsqrt(deg)
    dis_ref[...] = dis
    y_ref[...] = jnp.dot(x_ref[...], w_ref[...],
                         preferred_element_type=jnp.float32) * dis


_tc_a = pl.pallas_call(
    _tc_a_body,
    grid=(GRID,),
    in_specs=[
        pl.BlockSpec((RB, 1), lambda i: (i, 0)),
        pl.BlockSpec((RB, 1), lambda i: (i, 0)),
        pl.BlockSpec((RB, D), lambda i: (i, 0)),
        pl.BlockSpec((D, D), lambda i: (0, 0)),
    ],
    out_specs=[
        pl.BlockSpec((RB, 1), lambda i: (i, 0)),
        pl.BlockSpec((RB, D), lambda i: (i, 0)),
    ],
    out_shape=[
        jax.ShapeDtypeStruct((NP, 1), jnp.float32),
        jax.ShapeDtypeStruct((NP, D), jnp.float32),
    ],
)


def _tc_mid_body(a0_ref, a1_ref, dis_ref, b_ref, w_ref, y2_ref):
    dis = dis_ref[...]
    h = jnp.maximum((a0_ref[...] + a1_ref[...]) * dis + b_ref[...], 0.0)
    y2_ref[...] = jnp.dot(h, w_ref[...],
                          preferred_element_type=jnp.float32) * dis


_tc_mid = pl.pallas_call(
    _tc_mid_body,
    grid=(GRID,),
    in_specs=[
        pl.BlockSpec((RB, D), lambda i: (i, 0)),
        pl.BlockSpec((RB, D), lambda i: (i, 0)),
        pl.BlockSpec((RB, 1), lambda i: (i, 0)),
        pl.BlockSpec((1, D), lambda i: (0, 0)),
        pl.BlockSpec((D, D), lambda i: (0, 0)),
    ],
    out_specs=pl.BlockSpec((RB, D), lambda i: (i, 0)),
    out_shape=jax.ShapeDtypeStruct((NP, D), jnp.float32),
)


def _tc_fin_body(a0_ref, a1_ref, dis_ref, b_ref, batch_ref, wfc_ref, bfc_ref,
                 out_ref, sums, cnts):
    i = pl.program_id(0)

    @pl.when(i == 0)
    def _():
        sums[...] = jnp.zeros_like(sums)
        cnts[...] = jnp.zeros_like(cnts)

    h = jnp.maximum((a0_ref[...] + a1_ref[...]) * dis_ref[...] + b_ref[...],
                    0.0)
    seg = lax.broadcasted_iota(jnp.int32, (RB, NB), 1)
    p = (batch_ref[...] == seg).astype(jnp.float32)
    dn = (((0,), (0,)), ((), ()))
    sums[...] += lax.dot_general(p, h, dn, preferred_element_type=jnp.float32)
    cnts[...] += lax.dot_general(p, jnp.ones((RB, D), jnp.float32), dn,
                                 preferred_element_type=jnp.float32)

    @pl.when(i == GRID - 1)
    def _():
        pooled = sums[...] / jnp.maximum(cnts[...], 1.0)
        out_ref[...] = jnp.dot(pooled, wfc_ref[...],
                               preferred_element_type=jnp.float32) + bfc_ref[...]


_tc_fin = pl.pallas_call(
    _tc_fin_body,
    grid=(GRID,),
    in_specs=[
        pl.BlockSpec((RB, D), lambda i: (i, 0)),
        pl.BlockSpec((RB, D), lambda i: (i, 0)),
        pl.BlockSpec((RB, 1), lambda i: (i, 0)),
        pl.BlockSpec((1, D), lambda i: (0, 0)),
        pl.BlockSpec((RB, 1), lambda i: (i, 0)),
        pl.BlockSpec((D, 1), lambda i: (0, 0)),
        pl.BlockSpec((1, 1), lambda i: (0, 0)),
    ],
    out_specs=pl.BlockSpec((NB, 1), lambda i: (0, 0)),
    out_shape=jax.ShapeDtypeStruct((NB, 1), jnp.float32),
    scratch_shapes=[
        pltpu.VMEM((NB, D), jnp.float32),
        pltpu.VMEM((NB, D), jnp.float32),
    ],
)


def kernel(x, edge_index, batch, W1, b1, W2, b2, Wfc, bfc):
    x = x.astype(jnp.float32)
    src = edge_index[0].astype(jnp.int32)
    dst = edge_index[1].astype(jnp.int32)
    npadw = CH * CK - E // NW
    padw = (DUMMY + 1 + jnp.arange(npadw, dtype=jnp.int32))[None, :]
    padw = jnp.broadcast_to(padw, (NW, npadw))
    srcp = jnp.concatenate([src.reshape(NW, E // NW), padw], axis=1)
    srcp = srcp.reshape(NW, CH, CK)
    dstp = jnp.concatenate([dst.reshape(NW, E // NW), padw], axis=1)
    dstp = dstp.reshape(NW, CH, CK)
    xp = jnp.pad(x, ((0, NP - N), (0, 0)))
    batchp = jnp.pad(batch.astype(jnp.int32), (0, NP - N),
                     constant_values=NB).reshape(NP, 1)

    cnt = _deg_call(dstp)
    cnt0 = cnt[0].reshape(NP, 1)
    cnt1 = cnt[1].reshape(NP, 1)
    dis, y1 = _tc_a(cnt0, cnt1, xp, W1)
    p1 = _scat_call(y1, srcp, dstp)
    y2 = _tc_mid(p1[0], p1[1], dis, b1.reshape(1, D), W2)
    p2 = _scat_call(y2, srcp, dstp)
    out = _tc_fin(p2[0], p2[1], dis, b2.reshape(1, D), batchp, Wfc,
                  bfc.reshape(1, 1))
    return out.reshape(NB)


# speedup vs baseline: 30.6110x; 1.0478x over previous
import jax
import jax.numpy as jnp
from jax import lax
from jax.experimental import pallas as pl
from jax.experimental.pallas import tpu as pltpu
from jax.experimental.pallas import tpu_sc as plsc

N = 10000
NP = 10240
D = 128
NB = 64
E = 320000
NC, NS = 2, 16
NW = NC * NS
CK = 128
CH = 79
PH = 40
EP = NW * CH * CK
RPT = NP // NS
DUMMY = N
RB = 512
GRID = NP // RB

_MESH = plsc.VectorSubcoreMesh(core_axis_name="c", subcore_axis_name="s",
                               num_cores=NC, num_subcores=NS)


def _deg_body(dst_hbm, cnt_out, dst_v, ones_v, zb_v, cnt_sp):
    c = lax.axis_index("c")
    s = lax.axis_index("s")
    w = c * NS + s
    pltpu.sync_copy(dst_hbm.at[w], dst_v)
    z16 = jnp.zeros((16,), jnp.float32)
    o16 = jnp.ones((16,), jnp.float32)
    for i in range(RPT // 16):
        zb_v[pl.ds(16 * i, 16)] = z16
    for i in range(CK // 16):
        ones_v[pl.ds(16 * i, 16)] = o16
    pltpu.sync_copy(zb_v, cnt_sp.at[pl.ds(s * RPT, RPT)])
    plsc.subcore_barrier()
    for g in range(CH):
        pltpu.sync_copy(ones_v, cnt_sp.at[dst_v.at[g]], add=True)
    plsc.subcore_barrier()
    pltpu.sync_copy(cnt_sp.at[pl.ds(s * RPT, RPT)],
                    cnt_out.at[c, pl.ds(s * RPT, RPT)])


_deg_call = pl.kernel(
    _deg_body,
    out_type=jax.ShapeDtypeStruct((NC, NP), jnp.float32),
    mesh=_MESH,
    scratch_types=[
        pltpu.VMEM((CH, CK), jnp.int32),
        pltpu.VMEM((CK,), jnp.float32),
        pltpu.VMEM((RPT,), jnp.float32),
        pltpu.VMEM_SHARED((NP,), jnp.float32),
    ],
)


NBUF = 2


def _scat_body(y_hbm, src_hbm, dst_hbm, out_hbm, src_v, dst_v, rows0, rows1,
               acc_sp, gsem0, gsem1):
    rows = (rows0, rows1)
    gsems = (gsem0, gsem1)
    c = lax.axis_index("c")
    s = lax.axis_index("s")
    w = c * NS + s
    base = s * RPT

    @pl.when(c == 0)
    def _():
        pltpu.sync_copy(y_hbm.at[pl.ds(base, RPT)], acc_sp.at[pl.ds(base, RPT)])

    @pl.when(c != 0)
    def _():
        z16 = jnp.zeros((16,), jnp.float32)

        def _zrow(i, carry):
            for j in range(D // 16):
                rows0[i, pl.ds(16 * j, 16)] = z16
            return carry

        lax.fori_loop(0, CK, _zrow, 0)
        off = 0
        while off < RPT:
            n = min(CK, RPT - off)
            pltpu.sync_copy(rows0.at[pl.ds(0, n)],
                            acc_sp.at[pl.ds(base + off, n)])
            off += n

    plsc.subcore_barrier()

    done = 0
    while done < CH:
        n = min(PH, CH - done)
        pltpu.sync_copy(src_hbm.at[w, pl.ds(done, n)], src_v.at[pl.ds(0, n)])
        pltpu.sync_copy(dst_hbm.at[w, pl.ds(done, n)], dst_v.at[pl.ds(0, n)])
        gd = [None] * n
        for g in range(n + 1):
            if g < n:
                b = g % NBUF
                gd[g] = pltpu.async_copy(y_hbm.at[src_v.at[g]], rows[b],
                                         gsems[b])
            if g >= 1:
                h = g - 1
                gd[h].wait()
                pltpu.sync_copy(rows[h % NBUF], acc_sp.at[dst_v.at[h]],
                                add=True)
        done += n

    plsc.subcore_barrier()
    pltpu.sync_copy(acc_sp.at[pl.ds(base, RPT)],
                    out_hbm.at[c, pl.ds(base, RPT)])


_scat_call = pl.kernel(
    _scat_body,
    out_type=jax.ShapeDtypeStruct((NC, NP, D), jnp.float32),
    mesh=_MESH,
    scratch_types=[
        pltpu.VMEM((PH, CK), jnp.int32),
        pltpu.VMEM((PH, CK), jnp.int32),
        pltpu.VMEM((CK, D), jnp.float32),
        pltpu.VMEM((CK, D), jnp.float32),
        pltpu.VMEM_SHARED((NP, D), jnp.float32),
        pltpu.SemaphoreType.DMA,
        pltpu.SemaphoreType.DMA,
    ],
)


def _tc_a_body(cnt0_ref, cnt1_ref, x_ref, w_ref, dis_ref, y_ref):
    deg = cnt0_ref[...] + cnt1_ref[...] + 1.0
    dis = lax.rsqrt(deg)
    dis_ref[...] = dis
    y_ref[...] = jnp.dot(x_ref[...], w_ref[...],
                         preferred_element_type=jnp.float32) * dis


_tc_a = pl.pallas_call(
    _tc_a_body,
    grid=(GRID,),
    in_specs=[
        pl.BlockSpec((RB, 1), lambda i: (i, 0)),
        pl.BlockSpec((RB, 1), lambda i: (GRID + i, 0)),
        pl.BlockSpec((RB, D), lambda i: (i, 0)),
        pl.BlockSpec((D, D), lambda i: (0, 0)),
    ],
    out_specs=[
        pl.BlockSpec((RB, 1), lambda i: (i, 0)),
        pl.BlockSpec((RB, D), lambda i: (i, 0)),
    ],
    out_shape=[
        jax.ShapeDtypeStruct((NP, 1), jnp.float32),
        jax.ShapeDtypeStruct((NP, D), jnp.float32),
    ],
)


def _tc_mid_body(a0_ref, a1_ref, dis_ref, b_ref, w_ref, y2_ref):
    dis = dis_ref[...]
    h = jnp.maximum((a0_ref[0] + a1_ref[0]) * dis + b_ref[...], 0.0)
    y2_ref[...] = jnp.dot(h, w_ref[...],
                          preferred_element_type=jnp.float32) * dis


_tc_mid = pl.pallas_call(
    _tc_mid_body,
    grid=(GRID,),
    in_specs=[
        pl.BlockSpec((1, RB, D), lambda i: (0, i, 0)),
        pl.BlockSpec((1, RB, D), lambda i: (1, i, 0)),
        pl.BlockSpec((RB, 1), lambda i: (i, 0)),
        pl.BlockSpec((1, D), lambda i: (0, 0)),
        pl.BlockSpec((D, D), lambda i: (0, 0)),
    ],
    out_specs=pl.BlockSpec((RB, D), lambda i: (i, 0)),
    out_shape=jax.ShapeDtypeStruct((NP, D), jnp.float32),
)


def _tc_fin_body(a0_ref, a1_ref, dis_ref, b_ref, batch_ref, wfc_ref, bfc_ref,
                 out_ref, sums, cnts):
    i = pl.program_id(0)

    @pl.when(i == 0)
    def _():
        sums[...] = jnp.zeros_like(sums)
        cnts[...] = jnp.zeros_like(cnts)

    h = jnp.maximum((a0_ref[0] + a1_ref[0]) * dis_ref[...] + b_ref[...],
                    0.0)
    seg = lax.broadcasted_iota(jnp.int32, (RB, NB), 1)
    p = (batch_ref[...] == seg).astype(jnp.float32)
    dn = (((0,), (0,)), ((), ()))
    sums[...] += lax.dot_general(p, h, dn, preferred_element_type=jnp.float32)
    cnts[...] += lax.dot_general(p, jnp.ones((RB, D), jnp.float32), dn,
                                 preferred_element_type=jnp.float32)

    @pl.when(i == GRID - 1)
    def _():
        pooled = sums[...] / jnp.maximum(cnts[...], 1.0)
        out_ref[...] = jnp.dot(pooled, wfc_ref[...],
                               preferred_element_type=jnp.float32) + bfc_ref[...]


_tc_fin = pl.pallas_call(
    _tc_fin_body,
    grid=(GRID,),
    in_specs=[
        pl.BlockSpec((1, RB, D), lambda i: (0, i, 0)),
        pl.BlockSpec((1, RB, D), lambda i: (1, i, 0)),
        pl.BlockSpec((RB, 1), lambda i: (i, 0)),
        pl.BlockSpec((1, D), lambda i: (0, 0)),
        pl.BlockSpec((RB, 1), lambda i: (i, 0)),
        pl.BlockSpec((D, 1), lambda i: (0, 0)),
        pl.BlockSpec((1, 1), lambda i: (0, 0)),
    ],
    out_specs=pl.BlockSpec((NB, 1), lambda i: (0, 0)),
    out_shape=jax.ShapeDtypeStruct((NB, 1), jnp.float32),
    scratch_shapes=[
        pltpu.VMEM((NB, D), jnp.float32),
        pltpu.VMEM((NB, D), jnp.float32),
    ],
)


def kernel(x, edge_index, batch, W1, b1, W2, b2, Wfc, bfc):
    x = x.astype(jnp.float32)
    src = edge_index[0].astype(jnp.int32)
    dst = edge_index[1].astype(jnp.int32)
    npadw = CH * CK - E // NW
    padw = (DUMMY + 1 + jnp.arange(npadw, dtype=jnp.int32))[None, :]
    padw = jnp.broadcast_to(padw, (NW, npadw))
    srcp = jnp.concatenate([src.reshape(NW, E // NW), padw], axis=1)
    srcp = srcp.reshape(NW, CH, CK)
    dstp = jnp.concatenate([dst.reshape(NW, E // NW), padw], axis=1)
    dstp = dstp.reshape(NW, CH, CK)
    xp = jnp.pad(x, ((0, NP - N), (0, 0)))
    batchp = jnp.pad(batch.astype(jnp.int32), (0, NP - N),
                     constant_values=NB).reshape(NP, 1)

    cnt = _deg_call(dstp).reshape(NC * NP, 1)
    dis, y1 = _tc_a(cnt, cnt, xp, W1)
    p1 = _scat_call(y1, srcp, dstp)
    y2 = _tc_mid(p1, p1, dis, b1.reshape(1, D), W2)
    p2 = _scat_call(y2, srcp, dstp)
    out = _tc_fin(p2, p2, dis, b2.reshape(1, D), batchp, Wfc,
                  bfc.reshape(1, 1))
    return out.reshape(NB)


# speedup vs baseline: 31.1782x; 1.0185x over previous
import jax
import jax.numpy as jnp
from jax import lax
from jax.experimental import pallas as pl
from jax.experimental.pallas import tpu as pltpu
from jax.experimental.pallas import tpu_sc as plsc

N = 10000
NP = 10240
D = 128
NB = 64
E = 320000
NC, NS = 2, 16
NW = NC * NS
CK = 128
CPW = 80
NCH = NW * CPW
PH = 40
RPT = NP // NS
RB = 512
GRID = NP // RB

_MESH = plsc.VectorSubcoreMesh(core_axis_name="c", subcore_axis_name="s",
                               num_cores=NC, num_subcores=NS)


def _deg_body(ei_hbm, cnt_out, dst_v, ones_v, zb_v, cnt_sp):
    c = lax.axis_index("c")
    s = lax.axis_index("s")
    w = c * NS + s
    pltpu.sync_copy(ei_hbm.at[1, pl.ds(w * CPW, CPW)], dst_v)
    z16 = jnp.zeros((16,), jnp.float32)
    o16 = jnp.ones((16,), jnp.float32)
    for i in range(RPT // 16):
        zb_v[pl.ds(16 * i, 16)] = z16
    for i in range(CK // 16):
        ones_v[pl.ds(16 * i, 16)] = o16
    pltpu.sync_copy(zb_v, cnt_sp.at[pl.ds(s * RPT, RPT)])
    plsc.subcore_barrier()
    for g in range(CPW):
        pltpu.sync_copy(ones_v, cnt_sp.at[dst_v.at[g]], add=True)
    plsc.subcore_barrier()
    pltpu.sync_copy(cnt_sp.at[pl.ds(s * RPT, RPT)],
                    cnt_out.at[c, pl.ds(s * RPT, RPT)])


_deg_call = pl.kernel(
    _deg_body,
    out_type=jax.ShapeDtypeStruct((NC, NP), jnp.float32),
    mesh=_MESH,
    scratch_types=[
        pltpu.VMEM((CPW, CK), jnp.int32),
        pltpu.VMEM((CK,), jnp.float32),
        pltpu.VMEM((RPT,), jnp.float32),
        pltpu.VMEM_SHARED((NP,), jnp.float32),
    ],
)


NBUF = 2


def _scat_body(y_hbm, ei_hbm, out_hbm, src_v, dst_v, rows0, rows1,
               acc_sp, gsem0, gsem1):
    rows = (rows0, rows1)
    gsems = (gsem0, gsem1)
    c = lax.axis_index("c")
    s = lax.axis_index("s")
    w = c * NS + s
    base = s * RPT

    @pl.when(c == 0)
    def _():
        pltpu.sync_copy(y_hbm.at[pl.ds(base, RPT)], acc_sp.at[pl.ds(base, RPT)])

    @pl.when(c != 0)
    def _():
        z16 = jnp.zeros((16,), jnp.float32)

        def _zrow(i, carry):
            for j in range(D // 16):
                rows0[i, pl.ds(16 * j, 16)] = z16
            return carry

        lax.fori_loop(0, CK, _zrow, 0)
        off = 0
        while off < RPT:
            n = min(CK, RPT - off)
            pltpu.sync_copy(rows0.at[pl.ds(0, n)],
                            acc_sp.at[pl.ds(base + off, n)])
            off += n

    plsc.subcore_barrier()

    done = 0
    while done < CPW:
        n = min(PH, CPW - done)
        pltpu.sync_copy(ei_hbm.at[0, pl.ds(w * CPW + done, n)],
                        src_v.at[pl.ds(0, n)])
        pltpu.sync_copy(ei_hbm.at[1, pl.ds(w * CPW + done, n)],
                        dst_v.at[pl.ds(0, n)])
        gd = [None] * n
        for g in range(n + 1):
            if g < n:
                b = g % NBUF
                gd[g] = pltpu.async_copy(y_hbm.at[src_v.at[g]], rows[b],
                                         gsems[b])
            if g >= 1:
                h = g - 1
                gd[h].wait()
                pltpu.sync_copy(rows[h % NBUF], acc_sp.at[dst_v.at[h]],
                                add=True)
        done += n

    plsc.subcore_barrier()
    pltpu.sync_copy(acc_sp.at[pl.ds(base, RPT)],
                    out_hbm.at[c, pl.ds(base, RPT)])


_scat_call = pl.kernel(
    _scat_body,
    out_type=jax.ShapeDtypeStruct((NC, NP, D), jnp.float32),
    mesh=_MESH,
    scratch_types=[
        pltpu.VMEM((PH, CK), jnp.int32),
        pltpu.VMEM((PH, CK), jnp.int32),
        pltpu.VMEM((CK, D), jnp.float32),
        pltpu.VMEM((CK, D), jnp.float32),
        pltpu.VMEM_SHARED((NP, D), jnp.float32),
        pltpu.SemaphoreType.DMA,
        pltpu.SemaphoreType.DMA,
    ],
)


def _tc_a_body(cnt0_ref, cnt1_ref, x_ref, w_ref, dis_ref, y_ref):
    deg = cnt0_ref[...] + cnt1_ref[...] + 1.0
    dis = lax.rsqrt(deg)
    dis_ref[...] = dis
    y_ref[...] = jnp.dot(x_ref[...], w_ref[...],
                         preferred_element_type=jnp.float32) * dis


_tc_a = pl.pallas_call(
    _tc_a_body,
    grid=(GRID,),
    in_specs=[
        pl.BlockSpec((RB, 1), lambda i: (i, 0)),
        pl.BlockSpec((RB, 1), lambda i: (GRID + i, 0)),
        pl.BlockSpec((RB, D), lambda i: (i, 0)),
        pl.BlockSpec((D, D), lambda i: (0, 0)),
    ],
    out_specs=[
        pl.BlockSpec((RB, 1), lambda i: (i, 0)),
        pl.BlockSpec((RB, D), lambda i: (i, 0)),
    ],
    out_shape=[
        jax.ShapeDtypeStruct((NP, 1), jnp.float32),
        jax.ShapeDtypeStruct((NP, D), jnp.float32),
    ],
)


def _tc_mid_body(a0_ref, a1_ref, dis_ref, b_ref, w_ref, y2_ref):
    dis = dis_ref[...]
    h = jnp.maximum((a0_ref[0] + a1_ref[0]) * dis + b_ref[...], 0.0)
    y2_ref[...] = jnp.dot(h, w_ref[...],
                          preferred_element_type=jnp.float32) * dis


_tc_mid = pl.pallas_call(
    _tc_mid_body,
    grid=(GRID,),
    in_specs=[
        pl.BlockSpec((1, RB, D), lambda i: (0, i, 0)),
        pl.BlockSpec((1, RB, D), lambda i: (1, i, 0)),
        pl.BlockSpec((RB, 1), lambda i: (i, 0)),
        pl.BlockSpec((1, D), lambda i: (0, 0)),
        pl.BlockSpec((D, D), lambda i: (0, 0)),
    ],
    out_specs=pl.BlockSpec((RB, D), lambda i: (i, 0)),
    out_shape=jax.ShapeDtypeStruct((NP, D), jnp.float32),
)


def _tc_fin_body(a0_ref, a1_ref, dis_ref, b_ref, batch_ref, wfc_ref, bfc_ref,
                 out_ref, sums, cnts):
    i = pl.program_id(0)

    @pl.when(i == 0)
    def _():
        sums[...] = jnp.zeros_like(sums)
        cnts[...] = jnp.zeros_like(cnts)

    h = jnp.maximum((a0_ref[0] + a1_ref[0]) * dis_ref[...] + b_ref[...],
                    0.0)
    seg = lax.broadcasted_iota(jnp.int32, (RB, NB), 1)
    p = (batch_ref[...] == seg).astype(jnp.float32)
    dn = (((0,), (0,)), ((), ()))
    sums[...] += lax.dot_general(p, h, dn, preferred_element_type=jnp.float32)
    cnts[...] += lax.dot_general(p, jnp.ones((RB, D), jnp.float32), dn,
                                 preferred_element_type=jnp.float32)

    @pl.when(i == GRID - 1)
    def _():
        pooled = sums[...] / jnp.maximum(cnts[...], 1.0)
        out_ref[...] = jnp.dot(pooled, wfc_ref[...],
                               preferred_element_type=jnp.float32) + bfc_ref[...]


_tc_fin = pl.pallas_call(
    _tc_fin_body,
    grid=(GRID,),
    in_specs=[
        pl.BlockSpec((1, RB, D), lambda i: (0, i, 0)),
        pl.BlockSpec((1, RB, D), lambda i: (1, i, 0)),
        pl.BlockSpec((RB, 1), lambda i: (i, 0)),
        pl.BlockSpec((1, D), lambda i: (0, 0)),
        pl.BlockSpec((RB, 1), lambda i: (i, 0)),
        pl.BlockSpec((D, 1), lambda i: (0, 0)),
        pl.BlockSpec((1, 1), lambda i: (0, 0)),
    ],
    out_specs=pl.BlockSpec((NB, 1), lambda i: (0, 0)),
    out_shape=jax.ShapeDtypeStruct((NB, 1), jnp.float32),
    scratch_shapes=[
        pltpu.VMEM((NB, D), jnp.float32),
        pltpu.VMEM((NB, D), jnp.float32),
    ],
)


def kernel(x, edge_index, batch, W1, b1, W2, b2, Wfc, bfc):
    x = x.astype(jnp.float32)
    npad = NCH * CK - E
    junk = N + 1 + (jnp.arange(npad, dtype=jnp.int32) % (NP - N - 1))
    ei3 = jnp.concatenate(
        [edge_index.astype(jnp.int32),
         jnp.broadcast_to(junk[None, :], (2, npad))], axis=1,
    ).reshape(2, NCH, CK)
    xp = jnp.pad(x, ((0, NP - N), (0, 0)))
    batchp = jnp.pad(batch.astype(jnp.int32), (0, NP - N),
                     constant_values=NB).reshape(NP, 1)

    cnt = _deg_call(ei3).reshape(NC * NP, 1)
    dis, y1 = _tc_a(cnt, cnt, xp, W1)
    p1 = _scat_call(y1, ei3)
    y2 = _tc_mid(p1, p1, dis, b1.reshape(1, D), W2)
    p2 = _scat_call(y2, ei3)
    out = _tc_fin(p2, p2, dis, b2.reshape(1, D), batchp, Wfc,
                  bfc.reshape(1, 1))
    return out.reshape(NB)
